# baseline (speedup 1.0000x reference)
import jax
import jax.numpy as jnp
from jax.experimental import pallas as pl
from jax.experimental.pallas import tpu as pltpu

_N = 320000
_D = 128
_D_OUT = 128
_S = 128
_P = 2
_B = 10000
_W = 32


def _main_body(xp_ref, xn_ref, bp_ref, bn_ref, w_ref, b_ref,
               out_ref, acc_p, acc_n):
    i = pl.program_id(1)
    nsteps = pl.num_programs(1)

    @pl.when(i == 0)
    def _init():
        acc_p[...] = jnp.zeros_like(acc_p)
        acc_n[...] = jnp.zeros_like(acc_n)

    w = w_ref[...].astype(jnp.bfloat16)
    bias = b_ref[...]

    def accum(x_ref, ids_ref, acc):
        h = jnp.dot(x_ref[0].astype(jnp.bfloat16), w,
                    preferred_element_type=jnp.float32) + bias
        h = jnp.maximum(h, 0.0).astype(jnp.bfloat16)
        ids = ids_ref[0, 0]
        lo = ids[0, 0]
        hi = ids[0, _B - 1]
        base = jnp.minimum((lo // 8) * 8, _S - _W)
        in_window = (hi - base) < _W

        @pl.when(in_window)
        def _fast():
            rel = ids - base
            iota_w = jax.lax.broadcasted_iota(jnp.int32, (_W, _B), 0)
            onehot = (rel == iota_w).astype(jnp.bfloat16)
            pooled = jnp.dot(onehot, h, preferred_element_type=jnp.float32)
            acc[pl.ds(base, _W), :] += pooled

        @pl.when(jnp.logical_not(in_window))
        def _slow():
            iota = jax.lax.broadcasted_iota(jnp.int32, (_S, _B), 0)
            onehot = (ids == iota).astype(jnp.bfloat16)
            acc[...] += jnp.dot(onehot, h, preferred_element_type=jnp.float32)

    accum(xp_ref, bp_ref, acc_p)
    accum(xn_ref, bn_ref, acc_n)

    @pl.when(i == nsteps - 1)
    def _finish():
        out_ref[0, :, :_D] = acc_p[...]
        out_ref[0, :, _D:] = acc_n[...]


def _combine_body(part_ref, mw_ref, mb_ref, out_ref):
    cat = part_ref[0] + part_ref[1]
    out_ref[...] = (
        jnp.dot(cat, mw_ref[...], preferred_element_type=jnp.float32)
        + mb_ref[...]
    )


def kernel(x_p, x_n, x_p_batch, x_n_batch, emb_weight, emb_bias,
           mlp_weight, mlp_bias):
    nsteps = _N // (_P * _B)
    bp = x_p_batch.reshape(_P, nsteps, 1, _B)
    bn = x_n_batch.reshape(_P, nsteps, 1, _B)
    xp = x_p.reshape(_P, _N // _P, _D)
    xn = x_n.reshape(_P, _N // _P, _D)
    partials = pl.pallas_call(
        _main_body,
        grid=(_P, nsteps),
        in_specs=[
            pl.BlockSpec((1, _B, _D), lambda p, i: (p, i, 0)),
            pl.BlockSpec((1, _B, _D), lambda p, i: (p, i, 0)),
            pl.BlockSpec((1, 1, 1, _B), lambda p, i: (p, i, 0, 0)),
            pl.BlockSpec((1, 1, 1, _B), lambda p, i: (p, i, 0, 0)),
            pl.BlockSpec((_D, _D), lambda p, i: (0, 0)),
            pl.BlockSpec((1, _D), lambda p, i: (0, 0)),
        ],
        out_specs=pl.BlockSpec((1, _S, 2 * _D), lambda p, i: (p, 0, 0)),
        out_shape=jax.ShapeDtypeStruct((_P, _S, 2 * _D), jnp.float32),
        scratch_shapes=[
            pltpu.VMEM((_S, _D), jnp.float32),
            pltpu.VMEM((_S, _D), jnp.float32),
        ],
        compiler_params=pltpu.CompilerParams(
            dimension_semantics=("parallel", "arbitrary"),
        ),
    )(xp, xn, bp, bn, emb_weight, emb_bias.reshape(1, _D))

    return pl.pallas_call(
        _combine_body,
        out_shape=jax.ShapeDtypeStruct((_S, _D_OUT), jnp.float32),
    )(partials, mlp_weight, mlp_bias.reshape(1, _D_OUT))
